# baseline (speedup 1.0000x reference)
import functools

import jax
import jax.numpy as jnp
from jax.experimental import pallas as pl
from jax.experimental.pallas import tpu as pltpu

_NBUF = 4


def _fused_body(adj_hbm, xa0_hbm, w_ref, out_hbm, h_s, xa0_s, out_s,
                b0, b1, b2, b3, s0, s1, s2, s3, sio, *, feat, cm, nchunks):
    l = pl.program_id(0)
    bufs = (b0, b1, b2, b3)
    sems = (s0, s1, s2, s3)
    ws = w_ref[0, :feat]
    wn = w_ref[0, feat:2 * feat]
    b = w_ref[0, 2 * feat:2 * feat + 1]

    @pl.when(l == 0)
    def _load_xa0():
        pltpu.make_async_copy(xa0_hbm, xa0_s, sio).start()
        pltpu.make_async_copy(xa0_hbm, xa0_s, sio).wait()

    def copy(c, k):
        return pltpu.make_async_copy(
            adj_hbm.at[pl.ds(c * cm, cm), :], bufs[k], sems[k])

    for k in range(_NBUF):
        if k < nchunks:
            copy(k, k).start()

    def epilogue(prod, xs):
        s = prod[:, :feat]
        deg = jnp.clip(prod[:, feat:feat + 1], 1e-6, None)
        neigh = s / deg
        return (jnp.dot(xs, ws, preferred_element_type=jnp.float32,
                        precision=jax.lax.Precision.HIGHEST)
                + jnp.dot(neigh, wn, preferred_element_type=jnp.float32,
                          precision=jax.lax.Precision.HIGHEST)
                + b)

    def chunk(c, k):
        copy(c, k).wait()
        a = bufs[k][...].astype(jnp.bfloat16)
        base = pl.multiple_of(c * cm, cm)

        @pl.when(l == 0)
        def _layer1():
            prod = jnp.dot(a, xa0_s[...], preferred_element_type=jnp.float32)
            xs = xa0_s[pl.ds(base, cm), :feat].astype(jnp.float32)
            h = jnp.maximum(epilogue(prod, xs), 0.0)
            h_s[pl.ds(base, cm), :feat] = h.astype(jnp.bfloat16)
            h_s[pl.ds(base, cm), feat:feat + 1] = jnp.ones(
                (cm, 1), jnp.bfloat16)

        @pl.when(l == 1)
        def _layer2():
            prod = jnp.dot(a, h_s[...], preferred_element_type=jnp.float32)
            xs2 = h_s[pl.ds(base, cm), :feat].astype(jnp.float32)
            out_s[pl.ds(base, cm), :] = epilogue(prod, xs2)

        @pl.when(c + _NBUF < nchunks)
        def _prefetch():
            copy(c + _NBUF, k).start()

    def group(p, carry):
        c0 = p * _NBUF
        for k in range(_NBUF):

            @pl.when(c0 + k < nchunks)
            def _do(k=k):
                chunk(c0 + k, k)

        return carry

    jax.lax.fori_loop(0, (nchunks + _NBUF - 1) // _NBUF, group, 0)

    @pl.when(l == 1)
    def _flush_out():
        pltpu.make_async_copy(out_s, out_hbm, sio).start()
        pltpu.make_async_copy(out_s, out_hbm, sio).wait()


def _pick_cm(n):
    for c in (200, 128, 80, 64, 40, 32, 16, 8):
        if n % c == 0:
            return c
    return n


def kernel(fts, adj, W1, b1, W2, b2):
    n, feat = fts.shape
    cm = _pick_cm(n)
    xa0 = jnp.concatenate(
        [fts.astype(jnp.bfloat16), jnp.ones((n, 1), jnp.bfloat16)], axis=1)
    wpack = jnp.stack([
        jnp.concatenate([W1[:feat], W1[feat:], b1.reshape(1, feat)], axis=0),
        jnp.concatenate([W2[:feat], W2[feat:], b2.reshape(1, feat)], axis=0),
    ])
    body = functools.partial(_fused_body, feat=feat, cm=cm, nchunks=n // cm)
    return pl.pallas_call(
        body,
        grid=(2,),
        in_specs=[
            pl.BlockSpec(memory_space=pl.ANY),
            pl.BlockSpec(memory_space=pl.ANY),
            pl.BlockSpec((1, 2 * feat + 1, feat), lambda l: (l, 0, 0)),
        ],
        out_specs=pl.BlockSpec(memory_space=pl.ANY),
        out_shape=jax.ShapeDtypeStruct((n, feat), jnp.float32),
        scratch_shapes=[
            pltpu.VMEM((n, feat + 1), jnp.bfloat16),
            pltpu.VMEM((n, feat + 1), jnp.bfloat16),
            pltpu.VMEM((n, feat), jnp.float32),
            pltpu.VMEM((cm, n), jnp.float32),
            pltpu.VMEM((cm, n), jnp.float32),
            pltpu.VMEM((cm, n), jnp.float32),
            pltpu.VMEM((cm, n), jnp.float32),
            pltpu.SemaphoreType.DMA,
            pltpu.SemaphoreType.DMA,
            pltpu.SemaphoreType.DMA,
            pltpu.SemaphoreType.DMA,
            pltpu.SemaphoreType.DMA,
        ],
        compiler_params=pltpu.CompilerParams(
            dimension_semantics=("arbitrary",),
            vmem_limit_bytes=64 * 1024 * 1024,
        ),
    )(adj, xa0, wpack)


# speedup vs baseline: 1.0393x; 1.0393x over previous
import functools

import jax
import jax.numpy as jnp
from jax.experimental import pallas as pl
from jax.experimental.pallas import tpu as pltpu


def _fused_body(adj_hbm, xa0_hbm, w_ref, out_hbm, abf_hbm, h_s, xa0_s, out_s,
                f0, f1, g0, g1, sf0, sf1, sg0, sg1, sio,
                *, feat, cm, nchunks):
    l = pl.program_id(0)
    fbufs = (f0, f1)
    gbufs = (g0, g1)
    fsems = (sf0, sf1)
    gsems = (sg0, sg1)
    ws = w_ref[0, :feat]
    wn = w_ref[0, feat:2 * feat]
    b = w_ref[0, 2 * feat:2 * feat + 1]

    def fcopy(c, k):
        return pltpu.make_async_copy(
            adj_hbm.at[pl.ds(c * cm, cm), :], fbufs[k], fsems[k])

    def gput(c, k):
        return pltpu.make_async_copy(
            gbufs[k], abf_hbm.at[pl.ds(c * cm, cm), :], gsems[k])

    def gget(c, k):
        return pltpu.make_async_copy(
            abf_hbm.at[pl.ds(c * cm, cm), :], gbufs[k], gsems[k])

    def epilogue(prod, xs):
        s = prod[:, :feat]
        deg = jnp.clip(prod[:, feat:feat + 1], 1e-6, None)
        neigh = s / deg
        return (jnp.dot(xs, ws, preferred_element_type=jnp.float32,
                        precision=jax.lax.Precision.HIGHEST)
                + jnp.dot(neigh, wn, preferred_element_type=jnp.float32,
                          precision=jax.lax.Precision.HIGHEST)
                + b)

    @pl.when(l == 0)
    def _layer1():
        pltpu.make_async_copy(xa0_hbm, xa0_s, sio).start()
        pltpu.make_async_copy(xa0_hbm, xa0_s, sio).wait()
        fcopy(0, 0).start()
        if nchunks > 1:
            fcopy(1, 1).start()

        def chunk(c, k):
            fcopy(c, k).wait()

            @pl.when(c >= 2)
            def _drain_prev_put():
                gput(c - 2, k).wait()

            gbufs[k][...] = fbufs[k][...].astype(jnp.bfloat16)
            a = gbufs[k][...]
            base = pl.multiple_of(c * cm, cm)
            prod = jnp.dot(a, xa0_s[...], preferred_element_type=jnp.float32)
            xs = xa0_s[pl.ds(base, cm), :feat].astype(jnp.float32)
            h = jnp.maximum(epilogue(prod, xs), 0.0)
            h_s[pl.ds(base, cm), :feat] = h.astype(jnp.bfloat16)
            h_s[pl.ds(base, cm), feat:feat + 1] = jnp.ones(
                (cm, 1), jnp.bfloat16)
            gput(c, k).start()

            @pl.when(c + 2 < nchunks)
            def _prefetch():
                fcopy(c + 2, k).start()

        def pair(p, carry):
            c0 = 2 * p
            chunk(c0, 0)

            @pl.when(c0 + 1 < nchunks)
            def _odd():
                chunk(c0 + 1, 1)

            return carry

        jax.lax.fori_loop(0, (nchunks + 1) // 2, pair, 0)
        last0 = nchunks - 1 if (nchunks - 1) % 2 == 0 else nchunks - 2
        if last0 >= 0:
            gput(last0, 0).wait()
        last1 = nchunks - 1 if (nchunks - 1) % 2 == 1 else nchunks - 2
        if last1 >= 1:
            gput(last1, 1).wait()

    @pl.when(l == 1)
    def _layer2():
        gget(0, 0).start()
        if nchunks > 1:
            gget(1, 1).start()

        def chunk(c, k):
            gget(c, k).wait()
            a = gbufs[k][...]
            base = pl.multiple_of(c * cm, cm)
            prod = jnp.dot(a, h_s[...], preferred_element_type=jnp.float32)
            xs2 = h_s[pl.ds(base, cm), :feat].astype(jnp.float32)
            out_s[pl.ds(base, cm), :] = epilogue(prod, xs2)

            @pl.when(c + 2 < nchunks)
            def _prefetch():
                gget(c + 2, k).start()

        def pair(p, carry):
            c0 = 2 * p
            chunk(c0, 0)

            @pl.when(c0 + 1 < nchunks)
            def _odd():
                chunk(c0 + 1, 1)

            return carry

        jax.lax.fori_loop(0, (nchunks + 1) // 2, pair, 0)
        pltpu.make_async_copy(out_s, out_hbm, sio).start()
        pltpu.make_async_copy(out_s, out_hbm, sio).wait()


def _pick_cm(n):
    for c in (400, 256, 200, 128, 80, 64, 40, 32, 16, 8):
        if n % c == 0:
            return c
    return n


def kernel(fts, adj, W1, b1, W2, b2):
    n, feat = fts.shape
    cm = _pick_cm(n)
    xa0 = jnp.concatenate(
        [fts.astype(jnp.bfloat16), jnp.ones((n, 1), jnp.bfloat16)], axis=1)
    wpack = jnp.stack([
        jnp.concatenate([W1[:feat], W1[feat:], b1.reshape(1, feat)], axis=0),
        jnp.concatenate([W2[:feat], W2[feat:], b2.reshape(1, feat)], axis=0),
    ])
    body = functools.partial(_fused_body, feat=feat, cm=cm, nchunks=n // cm)
    return pl.pallas_call(
        body,
        grid=(2,),
        in_specs=[
            pl.BlockSpec(memory_space=pl.ANY),
            pl.BlockSpec(memory_space=pl.ANY),
            pl.BlockSpec((1, 2 * feat + 1, feat), lambda l: (l, 0, 0)),
        ],
        out_specs=(pl.BlockSpec(memory_space=pl.ANY),
                   pl.BlockSpec(memory_space=pl.ANY)),
        out_shape=(jax.ShapeDtypeStruct((n, feat), jnp.float32),
                   jax.ShapeDtypeStruct((n, n), jnp.bfloat16)),
        scratch_shapes=[
            pltpu.VMEM((n, feat + 1), jnp.bfloat16),
            pltpu.VMEM((n, feat + 1), jnp.bfloat16),
            pltpu.VMEM((n, feat), jnp.float32),
            pltpu.VMEM((cm, n), jnp.float32),
            pltpu.VMEM((cm, n), jnp.float32),
            pltpu.VMEM((cm, n), jnp.bfloat16),
            pltpu.VMEM((cm, n), jnp.bfloat16),
            pltpu.SemaphoreType.DMA,
            pltpu.SemaphoreType.DMA,
            pltpu.SemaphoreType.DMA,
            pltpu.SemaphoreType.DMA,
            pltpu.SemaphoreType.DMA,
        ],
        compiler_params=pltpu.CompilerParams(
            dimension_semantics=("arbitrary",),
            vmem_limit_bytes=64 * 1024 * 1024,
        ),
    )(adj, xa0, wpack)[0]


# speedup vs baseline: 1.0976x; 1.0561x over previous
import functools

import jax
import jax.numpy as jnp
from jax.experimental import pallas as pl
from jax.experimental.pallas import tpu as pltpu


def _fused_body(adj_ref, xa0_ref, w_ref, out_ref, h_s, *, feat, bm):
    l = pl.program_id(0)
    i = pl.program_id(1)
    a = adj_ref[...].astype(jnp.bfloat16)
    base = pl.multiple_of(i * bm, bm)
    ws = w_ref[0, :feat]
    wn = w_ref[0, feat:2 * feat]
    b = w_ref[0, 2 * feat:2 * feat + 1]

    def _epilogue(prod, xs):
        s = prod[:, :feat]
        deg = jnp.clip(prod[:, feat:feat + 1], 1e-6, None)
        neigh = s / deg
        return (jnp.dot(xs, ws, preferred_element_type=jnp.float32,
                        precision=jax.lax.Precision.HIGHEST)
                + jnp.dot(neigh, wn, preferred_element_type=jnp.float32,
                          precision=jax.lax.Precision.HIGHEST)
                + b)

    @pl.when(l == 0)
    def _layer1():
        prod = jnp.dot(a, xa0_ref[...], preferred_element_type=jnp.float32)
        xs = xa0_ref[pl.ds(base, bm), :feat].astype(jnp.float32)
        h = jnp.maximum(_epilogue(prod, xs), 0.0)
        h_s[pl.ds(base, bm), :feat] = h.astype(jnp.bfloat16)
        h_s[pl.ds(base, bm), feat:feat + 1] = jnp.ones((bm, 1), jnp.bfloat16)
        out_ref[...] = h

    @pl.when(l == 1)
    def _layer2():
        prod = jnp.dot(a, h_s[...], preferred_element_type=jnp.float32)
        xs2 = h_s[pl.ds(base, bm), :feat].astype(jnp.float32)
        out_ref[...] = _epilogue(prod, xs2)


def _pick_bm(n):
    for c in (400, 256, 200, 128, 80, 64, 40, 32, 16, 8):
        if n % c == 0:
            return c
    return n


def kernel(fts, adj, W1, b1, W2, b2):
    n, feat = fts.shape
    bm = _pick_bm(n)
    xa0 = jnp.concatenate(
        [fts.astype(jnp.bfloat16), jnp.ones((n, 1), jnp.bfloat16)], axis=1)
    wpack = jnp.stack([
        jnp.concatenate([W1[:feat], W1[feat:], b1.reshape(1, feat)], axis=0),
        jnp.concatenate([W2[:feat], W2[feat:], b2.reshape(1, feat)], axis=0),
    ])
    body = functools.partial(_fused_body, feat=feat, bm=bm)
    return pl.pallas_call(
        body,
        grid=(2, n // bm),
        in_specs=[
            pl.BlockSpec((bm, n), lambda l, i: (i, 0)),
            pl.BlockSpec((n, feat + 1), lambda l, i: (0, 0)),
            pl.BlockSpec((1, 2 * feat + 1, feat), lambda l, i: (l, 0, 0)),
        ],
        out_specs=pl.BlockSpec((bm, feat), lambda l, i: (i * l, 0)),
        out_shape=jax.ShapeDtypeStruct((n, feat), jnp.float32),
        scratch_shapes=[pltpu.VMEM((n, feat + 1), jnp.bfloat16)],
        compiler_params=pltpu.CompilerParams(
            dimension_semantics=("arbitrary", "arbitrary"),
        ),
    )(adj, xa0, wpack)
